# Initial kernel scaffold; baseline (speedup 1.0000x reference)
import jax
import jax.numpy as jnp
from jax.experimental import pallas as pl


def kernel(instance_embs, memory_bank):
    raise NotImplementedError("write your pallas kernel here")


# speedup vs baseline: 1.2925x; 1.2925x over previous
import jax
import jax.numpy as jnp
from jax.experimental import pallas as pl
from jax.experimental.pallas import tpu as pltpu

WAY = 5
SHOT = 5
QUERY = 15
EMB_DIM = 640
N_MEMORY = 100000
TEMPERATURE = 64.0
TOPK = 8
EPS = 1e-12

NSUP = WAY * SHOT
NQ = WAY * QUERY
BM = 2000
NB = N_MEMORY // BM
WPAD = 8
NEG = -1e30
BIG = 1 << 30


def _mean_support_dirs(inst_ref, rows):
    e25 = inst_ref[0:NSUP, :]
    ss = jnp.sum(e25 * e25, axis=1, keepdims=True)
    en = e25 / jnp.maximum(jnp.sqrt(ss), EPS)
    wio = jax.lax.broadcasted_iota(jnp.int32, (rows, NSUP), 0)
    jio = jax.lax.broadcasted_iota(jnp.int32, (rows, NSUP), 1)
    msel = jnp.where(jio % WAY == wio, 1.0 / SHOT, 0.0)
    a = jax.lax.dot_general(msel, en, (((1,), (0,)), ((), ())),
                            preferred_element_type=jnp.float32)
    return e25, en, a, wio, jio


def _scan_kernel(inst_ref, mem_ref, vals_ref, idx_ref):
    i = pl.program_id(0)
    _, _, a, _, _ = _mean_support_dirs(inst_ref, WAY)
    x = mem_ref[...]
    s = jax.lax.dot_general(a, x, (((1,), (1,)), ((), ())),
                            preferred_element_type=jnp.float32)
    ones = jnp.ones((1, EMB_DIM), jnp.float32)
    sq = jax.lax.dot_general(ones, x * x, (((1,), (1,)), ((), ())),
                             preferred_element_type=jnp.float32)
    rn = 1.0 / jnp.maximum(jnp.sqrt(sq), EPS)
    sim = s * rn

    lio = jax.lax.broadcasted_iota(jnp.int32, (WAY, BM), 1)
    cur = sim
    vlist, ilist = [], []
    for _ in range(TOPK):
        mx = jnp.max(cur, axis=1, keepdims=True)
        p = jnp.min(jnp.where(cur >= mx, lio, BIG), axis=1, keepdims=True)
        vlist.append(mx)
        ilist.append(p)
        cur = jnp.where(lio == p, NEG, cur)
    vals8 = jnp.concatenate(vlist, axis=1)
    idx8 = jnp.concatenate(ilist, axis=1) + i * BM
    vals_ref[0, 0:WAY, :] = vals8
    vals_ref[0, WAY:WPAD, :] = jnp.full((WPAD - WAY, TOPK), NEG, jnp.float32)
    idx_ref[0, 0:WAY, :] = idx8
    idx_ref[0, WAY:WPAD, :] = jnp.zeros((WPAD - WAY, TOPK), jnp.int32)


def _merge_kernel(vals_ref, idx_ref, inst_ref, gi_ref, wmn_ref, psup_ref):
    e25, en, a, wio, jio = _mean_support_dirs(inst_ref, WPAD)
    g = jax.lax.dot_general(a, en, (((1,), (1,)), ((), ())),
                            preferred_element_type=jnp.float32)
    gm = jnp.where(jio % WAY == wio, g, NEG)
    cand = jnp.concatenate([gm, vals_ref[...]], axis=1)
    nc = NSUP + NB * TOPK
    midxf = idx_ref[...].astype(jnp.float32)
    lio = jax.lax.broadcasted_iota(jnp.int32, (WPAD, nc), 1)

    cur = cand
    wsel = jnp.zeros((WPAD, nc), jnp.float32)
    glist, wlist = [], []
    for _ in range(TOPK):
        mx = jnp.max(cur, axis=1, keepdims=True)
        p = jnp.min(jnp.where(cur >= mx, lio, BIG), axis=1, keepdims=True)
        oh = lio == p
        wsel = wsel + jnp.where(oh, mx, 0.0)
        ohm = oh[:, NSUP:]
        is_mem = jnp.sum(jnp.where(ohm, 1.0, 0.0), axis=1, keepdims=True)
        glist.append(jnp.sum(jnp.where(ohm, midxf, 0.0), axis=1, keepdims=True))
        wlist.append(mx * is_mem)
        cur = jnp.where(oh, NEG, cur)

    rden = 1.0 / jnp.sum(wsel, axis=1, keepdims=True)
    gi_ref[...] = jnp.concatenate(glist, axis=1).astype(jnp.int32)
    wmn_ref[...] = jnp.concatenate(wlist, axis=1) * rden
    psup_ref[...] = jax.lax.dot_general(
        wsel[:, 0:NSUP] * rden, e25, (((1,), (0,)), ((), ())),
        preferred_element_type=jnp.float32)


def _finalize_kernel(gf_ref, wmn_ref, psup_ref, inst_ref, mem_ref, out_ref,
                     rows_ref, sems_ref):
    def _cp(k):
        return pltpu.make_async_copy(
            mem_ref.at[pl.ds(gf_ref[k], 1), :],
            rows_ref.at[pl.ds(k, 1), :],
            sems_ref.at[k])

    for k in range(WAY * TOPK):
        _cp(k).start()
    for k in range(WAY * TOPK):
        _cp(k).wait()

    parts = []
    for w in range(WAY):
        rw = rows_ref[w * TOPK:(w + 1) * TOPK, :]
        ww = wmn_ref[w:w + 1, :]
        parts.append(jax.lax.dot_general(
            ww, rw, (((1,), (0,)), ((), ())),
            preferred_element_type=jnp.float32))
    pmem = jnp.concatenate(parts, axis=0)
    p = psup_ref[0:WAY, :] + pmem
    pn = p / jnp.maximum(jnp.sqrt(jnp.sum(p * p, axis=1, keepdims=True)), EPS)
    q = inst_ref[NSUP:NSUP + NQ, :]
    out_ref[...] = jax.lax.dot_general(
        q, pn, (((1,), (1,)), ((), ())),
        preferred_element_type=jnp.float32) * (1.0 / TEMPERATURE)


def kernel(instance_embs, memory_bank):
    vals, idx = pl.pallas_call(
        _scan_kernel,
        grid=(NB,),
        in_specs=[
            pl.BlockSpec((NSUP + NQ, EMB_DIM), lambda i: (0, 0)),
            pl.BlockSpec((BM, EMB_DIM), lambda i: (i, 0)),
        ],
        out_specs=[
            pl.BlockSpec((1, WPAD, TOPK), lambda i: (i, 0, 0)),
            pl.BlockSpec((1, WPAD, TOPK), lambda i: (i, 0, 0)),
        ],
        out_shape=[
            jax.ShapeDtypeStruct((NB, WPAD, TOPK), jnp.float32),
            jax.ShapeDtypeStruct((NB, WPAD, TOPK), jnp.int32),
        ],
        compiler_params=pltpu.CompilerParams(
            dimension_semantics=("parallel",),
            vmem_limit_bytes=48 * 1024 * 1024,
        ),
        name="proto_scan",
    )(instance_embs, memory_bank)

    vals_t = vals.transpose(1, 0, 2).reshape(WPAD, NB * TOPK)
    idx_t = idx.transpose(1, 0, 2).reshape(WPAD, NB * TOPK)

    gi, wmn, psup = pl.pallas_call(
        _merge_kernel,
        out_shape=[
            jax.ShapeDtypeStruct((WPAD, TOPK), jnp.int32),
            jax.ShapeDtypeStruct((WPAD, TOPK), jnp.float32),
            jax.ShapeDtypeStruct((WPAD, EMB_DIM), jnp.float32),
        ],
        name="proto_merge",
    )(vals_t, idx_t, instance_embs)

    gflat = gi[0:WAY].reshape(WAY * TOPK)

    logits = pl.pallas_call(
        _finalize_kernel,
        in_specs=[
            pl.BlockSpec(memory_space=pltpu.SMEM),
            pl.BlockSpec(memory_space=pltpu.VMEM),
            pl.BlockSpec(memory_space=pltpu.VMEM),
            pl.BlockSpec(memory_space=pltpu.VMEM),
            pl.BlockSpec(memory_space=pl.ANY),
        ],
        out_shape=jax.ShapeDtypeStruct((NQ, WAY), jnp.float32),
        scratch_shapes=[
            pltpu.VMEM((WAY * TOPK, EMB_DIM), jnp.float32),
            pltpu.SemaphoreType.DMA((WAY * TOPK,)),
        ],
        name="proto_finalize",
    )(gflat, wmn, psup, instance_embs, memory_bank)

    return logits


# speedup vs baseline: 2.0265x; 1.5678x over previous
import jax
import jax.numpy as jnp
from jax.experimental import pallas as pl
from jax.experimental.pallas import tpu as pltpu

WAY = 5
SHOT = 5
QUERY = 15
EMB_DIM = 640
N_MEMORY = 100000
TEMPERATURE = 64.0
TOPK = 8
EPS = 1e-12

NSUP = WAY * SHOT
NQ = WAY * QUERY
BM = 5000
NB = N_MEMORY // BM
CHB = (0, 1280, 2560, 3840, 5000)
NCH = len(CHB) - 1
CPB = NCH * TOPK
WPAD = 8
NEG = -1e30
BIG = 1 << 30


def _mean_support_dirs(inst_ref, rows):
    e25 = inst_ref[0:NSUP, :]
    ss = jnp.sum(e25 * e25, axis=1, keepdims=True)
    en = e25 / jnp.maximum(jnp.sqrt(ss), EPS)
    wio = jax.lax.broadcasted_iota(jnp.int32, (rows, NSUP), 0)
    jio = jax.lax.broadcasted_iota(jnp.int32, (rows, NSUP), 1)
    msel = jnp.where(jio % WAY == wio, 1.0 / SHOT, 0.0)
    a = jax.lax.dot_general(msel, en, (((1,), (0,)), ((), ())),
                            preferred_element_type=jnp.float32)
    return e25, en, a, wio, jio


def _scan_kernel(inst_ref, mem_ref, vals_ref, idx_ref):
    i = pl.program_id(0)
    _, _, a, _, _ = _mean_support_dirs(inst_ref, WAY)
    ones = jnp.ones((1, EMB_DIM), jnp.float32)
    vlist, ilist = [], []
    for q in range(NCH):
        s0, e0 = CHB[q], CHB[q + 1]
        ln = e0 - s0
        xc = mem_ref[s0:e0, :]
        sc = jax.lax.dot_general(a, xc, (((1,), (1,)), ((), ())),
                                 preferred_element_type=jnp.float32)
        sqc = jax.lax.dot_general(ones, xc * xc, (((1,), (1,)), ((), ())),
                                  preferred_element_type=jnp.float32)
        rnc = 1.0 / jnp.maximum(jnp.sqrt(sqc), EPS)
        sim = sc * rnc
        lio = jax.lax.broadcasted_iota(jnp.int32, (WAY, ln), 1)
        cur = sim
        for _ in range(TOPK):
            mx = jnp.max(cur, axis=1, keepdims=True)
            p = jnp.min(jnp.where(cur >= mx, lio, BIG), axis=1, keepdims=True)
            vlist.append(mx)
            ilist.append(p + (i * BM + s0))
            cur = jnp.where(lio == p, NEG, cur)
    vals32 = jnp.concatenate(vlist, axis=1)
    idx32 = jnp.concatenate(ilist, axis=1)
    vals_ref[0, 0:WAY, :] = vals32
    vals_ref[0, WAY:WPAD, :] = jnp.full((WPAD - WAY, CPB), NEG, jnp.float32)
    idx_ref[0, 0:WAY, :] = idx32
    idx_ref[0, WAY:WPAD, :] = jnp.zeros((WPAD - WAY, CPB), jnp.int32)


def _merge_kernel(vals_ref, idx_ref, inst_ref, gi_ref, wmn_ref, psup_ref):
    e25, en, a, wio, jio = _mean_support_dirs(inst_ref, WPAD)
    g = jax.lax.dot_general(a, en, (((1,), (1,)), ((), ())),
                            preferred_element_type=jnp.float32)
    gm = jnp.where(jio % WAY == wio, g, NEG)
    cand = jnp.concatenate([gm, vals_ref[...]], axis=1)
    nc = NSUP + NB * CPB
    midxf = idx_ref[...].astype(jnp.float32)
    lio = jax.lax.broadcasted_iota(jnp.int32, (WPAD, nc), 1)

    cur = cand
    wsel = jnp.zeros((WPAD, nc), jnp.float32)
    glist, wlist = [], []
    for _ in range(TOPK):
        mx = jnp.max(cur, axis=1, keepdims=True)
        p = jnp.min(jnp.where(cur >= mx, lio, BIG), axis=1, keepdims=True)
        oh = lio == p
        wsel = wsel + jnp.where(oh, mx, 0.0)
        ohm = oh[:, NSUP:]
        is_mem = jnp.sum(jnp.where(ohm, 1.0, 0.0), axis=1, keepdims=True)
        glist.append(jnp.sum(jnp.where(ohm, midxf, 0.0), axis=1, keepdims=True))
        wlist.append(mx * is_mem)
        cur = jnp.where(oh, NEG, cur)

    rden = 1.0 / jnp.sum(wsel, axis=1, keepdims=True)
    gi_ref[...] = jnp.concatenate(glist, axis=1).astype(jnp.int32)
    wmn_ref[...] = jnp.concatenate(wlist, axis=1) * rden
    psup_ref[...] = jax.lax.dot_general(
        wsel[:, 0:NSUP] * rden, e25, (((1,), (0,)), ((), ())),
        preferred_element_type=jnp.float32)


def _finalize_kernel(gf_ref, wmn_ref, psup_ref, inst_ref, mem_ref, out_ref,
                     rows_ref, sems_ref):
    def _cp(k):
        return pltpu.make_async_copy(
            mem_ref.at[pl.ds(gf_ref[k], 1), :],
            rows_ref.at[pl.ds(k, 1), :],
            sems_ref.at[k])

    for k in range(WAY * TOPK):
        _cp(k).start()
    for k in range(WAY * TOPK):
        _cp(k).wait()

    parts = []
    for w in range(WAY):
        rw = rows_ref[w * TOPK:(w + 1) * TOPK, :]
        ww = wmn_ref[w:w + 1, :]
        parts.append(jax.lax.dot_general(
            ww, rw, (((1,), (0,)), ((), ())),
            preferred_element_type=jnp.float32))
    pmem = jnp.concatenate(parts, axis=0)
    p = psup_ref[0:WAY, :] + pmem
    pn = p / jnp.maximum(jnp.sqrt(jnp.sum(p * p, axis=1, keepdims=True)), EPS)
    q = inst_ref[NSUP:NSUP + NQ, :]
    out_ref[...] = jax.lax.dot_general(
        q, pn, (((1,), (1,)), ((), ())),
        preferred_element_type=jnp.float32) * (1.0 / TEMPERATURE)


def kernel(instance_embs, memory_bank):
    vals, idx = pl.pallas_call(
        _scan_kernel,
        grid=(NB,),
        in_specs=[
            pl.BlockSpec((NSUP + NQ, EMB_DIM), lambda i: (0, 0)),
            pl.BlockSpec((BM, EMB_DIM), lambda i: (i, 0)),
        ],
        out_specs=[
            pl.BlockSpec((1, WPAD, CPB), lambda i: (i, 0, 0)),
            pl.BlockSpec((1, WPAD, CPB), lambda i: (i, 0, 0)),
        ],
        out_shape=[
            jax.ShapeDtypeStruct((NB, WPAD, CPB), jnp.float32),
            jax.ShapeDtypeStruct((NB, WPAD, CPB), jnp.int32),
        ],
        compiler_params=pltpu.CompilerParams(
            dimension_semantics=("arbitrary",),
            vmem_limit_bytes=50 * 1024 * 1024,
        ),
        name="proto_scan",
    )(instance_embs, memory_bank)

    vals_t = vals.transpose(1, 0, 2).reshape(WPAD, NB * CPB)
    idx_t = idx.transpose(1, 0, 2).reshape(WPAD, NB * CPB)

    gi, wmn, psup = pl.pallas_call(
        _merge_kernel,
        out_shape=[
            jax.ShapeDtypeStruct((WPAD, TOPK), jnp.int32),
            jax.ShapeDtypeStruct((WPAD, TOPK), jnp.float32),
            jax.ShapeDtypeStruct((WPAD, EMB_DIM), jnp.float32),
        ],
        name="proto_merge",
    )(vals_t, idx_t, instance_embs)

    gflat = gi[0:WAY].reshape(WAY * TOPK)

    logits = pl.pallas_call(
        _finalize_kernel,
        in_specs=[
            pl.BlockSpec(memory_space=pltpu.SMEM),
            pl.BlockSpec(memory_space=pltpu.VMEM),
            pl.BlockSpec(memory_space=pltpu.VMEM),
            pl.BlockSpec(memory_space=pltpu.VMEM),
            pl.BlockSpec(memory_space=pl.ANY),
        ],
        out_shape=jax.ShapeDtypeStruct((NQ, WAY), jnp.float32),
        scratch_shapes=[
            pltpu.VMEM((WAY * TOPK, EMB_DIM), jnp.float32),
            pltpu.SemaphoreType.DMA((WAY * TOPK,)),
        ],
        name="proto_finalize",
    )(gflat, wmn, psup, instance_embs, memory_bank)

    return logits


# speedup vs baseline: 2.0492x; 1.0112x over previous
import jax
import jax.numpy as jnp
from jax.experimental import pallas as pl
from jax.experimental.pallas import tpu as pltpu

WAY = 5
SHOT = 5
QUERY = 15
EMB_DIM = 640
N_MEMORY = 100000
TEMPERATURE = 64.0
TOPK = 8
EPS = 1e-12

NSUP = WAY * SHOT
NQ = WAY * QUERY
BM = 5000
NB = N_MEMORY // BM
CHB = (0, 1280, 2560, 3840, 5000)
NCH = len(CHB) - 1
CPB = NCH * TOPK
LPB = 128
WPAD = 8
NEG = -1e30
BIG = 1 << 30
NWIN = WAY * TOPK


def _mean_support_dirs(inst_ref, rows):
    e25 = inst_ref[0:NSUP, :]
    ss = jnp.sum(e25 * e25, axis=1, keepdims=True)
    en = e25 / jnp.maximum(jnp.sqrt(ss), EPS)
    wio = jax.lax.broadcasted_iota(jnp.int32, (rows, NSUP), 0)
    jio = jax.lax.broadcasted_iota(jnp.int32, (rows, NSUP), 1)
    msel = jnp.where(jio % WAY == wio, 1.0 / SHOT, 0.0)
    a = jax.lax.dot_general(msel, en, (((1,), (0,)), ((), ())),
                            preferred_element_type=jnp.float32)
    return e25, en, a, wio, jio


def _scan_kernel(inst_ref, mem_ref, vals_ref, idx_ref):
    i = pl.program_id(0)
    _, _, a, _, _ = _mean_support_dirs(inst_ref, WAY)
    ones = jnp.ones((1, EMB_DIM), jnp.float32)
    vlist, ilist = [], []
    for q in range(NCH):
        s0, e0 = CHB[q], CHB[q + 1]
        ln = e0 - s0
        xc = mem_ref[s0:e0, :]
        sc = jax.lax.dot_general(a, xc, (((1,), (1,)), ((), ())),
                                 preferred_element_type=jnp.float32)
        sqc = jax.lax.dot_general(ones, xc * xc, (((1,), (1,)), ((), ())),
                                  preferred_element_type=jnp.float32)
        rnc = 1.0 / jnp.maximum(jnp.sqrt(sqc), EPS)
        sim = sc * rnc
        lio = jax.lax.broadcasted_iota(jnp.int32, (WAY, ln), 1)
        cur = sim
        for _ in range(TOPK):
            mx = jnp.max(cur, axis=1, keepdims=True)
            p = jnp.min(jnp.where(cur >= mx, lio, BIG), axis=1, keepdims=True)
            vlist.append(mx)
            ilist.append(p + (i * BM + s0))
            cur = jnp.where(lio == p, NEG, cur)
    vlist.append(jnp.full((WAY, LPB - CPB), NEG, jnp.float32))
    ilist.append(jnp.zeros((WAY, LPB - CPB), jnp.int32))
    vals = jnp.concatenate(vlist, axis=1)
    idx = jnp.concatenate(ilist, axis=1)
    vals_ref[0:WAY, :] = vals
    vals_ref[WAY:WPAD, :] = jnp.full((WPAD - WAY, LPB), NEG, jnp.float32)
    idx_ref[0:WAY, :] = idx
    idx_ref[WAY:WPAD, :] = jnp.zeros((WPAD - WAY, LPB), jnp.int32)


def _tail_kernel(vals_ref, idx_ref, inst_ref, mem_ref, out_ref,
                 rows_ref, sems_ref):
    e25, en, a, wio, jio = _mean_support_dirs(inst_ref, WPAD)
    g = jax.lax.dot_general(a, en, (((1,), (1,)), ((), ())),
                            preferred_element_type=jnp.float32)
    gm = jnp.where(jio % WAY == wio, g, NEG)
    cand = jnp.concatenate([gm, vals_ref[...]], axis=1)
    nc = NSUP + NB * LPB
    midxf = idx_ref[...].astype(jnp.float32)
    lio = jax.lax.broadcasted_iota(jnp.int32, (WPAD, nc), 1)

    cur = cand
    wsel = jnp.zeros((WPAD, nc), jnp.float32)
    glist, wlist = [], []
    for _ in range(TOPK):
        mx = jnp.max(cur, axis=1, keepdims=True)
        p = jnp.min(jnp.where(cur >= mx, lio, BIG), axis=1, keepdims=True)
        oh = lio == p
        wsel = wsel + jnp.where(oh, mx, 0.0)
        ohm = oh[:, NSUP:]
        is_mem = jnp.sum(jnp.where(ohm, 1.0, 0.0), axis=1, keepdims=True)
        glist.append(jnp.sum(jnp.where(ohm, midxf, 0.0), axis=1, keepdims=True))
        wlist.append(mx * is_mem)
        cur = jnp.where(oh, NEG, cur)

    rden = 1.0 / jnp.sum(wsel, axis=1, keepdims=True)
    gi = jnp.concatenate(glist, axis=1).astype(jnp.int32)
    wmn = jnp.concatenate(wlist, axis=1) * rden
    psup = jax.lax.dot_general(
        wsel[:, 0:NSUP] * rden, e25, (((1,), (0,)), ((), ())),
        preferred_element_type=jnp.float32)

    def _cp(k):
        return pltpu.make_async_copy(
            mem_ref.at[pl.ds(gi[k // TOPK, k % TOPK], 1), :],
            rows_ref.at[pl.ds(k, 1), :],
            sems_ref.at[k])

    for k in range(NWIN):
        _cp(k).start()
    for k in range(NWIN):
        _cp(k).wait()

    parts = []
    for w in range(WAY):
        rw = rows_ref[w * TOPK:(w + 1) * TOPK, :]
        ww = wmn[w:w + 1, :]
        parts.append(jax.lax.dot_general(
            ww, rw, (((1,), (0,)), ((), ())),
            preferred_element_type=jnp.float32))
    p5 = psup[0:WAY, :] + jnp.concatenate(parts, axis=0)
    pn = p5 / jnp.maximum(
        jnp.sqrt(jnp.sum(p5 * p5, axis=1, keepdims=True)), EPS)
    q = inst_ref[NSUP:NSUP + NQ, :]
    out_ref[...] = jax.lax.dot_general(
        q, pn, (((1,), (1,)), ((), ())),
        preferred_element_type=jnp.float32) * (1.0 / TEMPERATURE)


def kernel(instance_embs, memory_bank):
    vals, idx = pl.pallas_call(
        _scan_kernel,
        grid=(NB,),
        in_specs=[
            pl.BlockSpec((NSUP + NQ, EMB_DIM), lambda i: (0, 0)),
            pl.BlockSpec((BM, EMB_DIM), lambda i: (i, 0)),
        ],
        out_specs=[
            pl.BlockSpec((WPAD, LPB), lambda i: (0, i)),
            pl.BlockSpec((WPAD, LPB), lambda i: (0, i)),
        ],
        out_shape=[
            jax.ShapeDtypeStruct((WPAD, NB * LPB), jnp.float32),
            jax.ShapeDtypeStruct((WPAD, NB * LPB), jnp.int32),
        ],
        compiler_params=pltpu.CompilerParams(
            dimension_semantics=("arbitrary",),
            vmem_limit_bytes=50 * 1024 * 1024,
        ),
        name="proto_scan",
    )(instance_embs, memory_bank)

    logits = pl.pallas_call(
        _tail_kernel,
        in_specs=[
            pl.BlockSpec(memory_space=pltpu.VMEM),
            pl.BlockSpec(memory_space=pltpu.VMEM),
            pl.BlockSpec(memory_space=pltpu.VMEM),
            pl.BlockSpec(memory_space=pl.ANY),
        ],
        out_shape=jax.ShapeDtypeStruct((NQ, WAY), jnp.float32),
        scratch_shapes=[
            pltpu.VMEM((NWIN, EMB_DIM), jnp.float32),
            pltpu.SemaphoreType.DMA((NWIN,)),
        ],
        name="proto_tail",
    )(vals, idx, instance_embs, memory_bank)

    return logits


# speedup vs baseline: 2.4848x; 1.2126x over previous
import jax
import jax.numpy as jnp
from jax.experimental import pallas as pl
from jax.experimental.pallas import tpu as pltpu

WAY = 5
SHOT = 5
QUERY = 15
EMB_DIM = 640
N_MEMORY = 100000
TEMPERATURE = 64.0
TOPK = 8
EPS = 1e-12

NSUP = WAY * SHOT
NQ = WAY * QUERY
BM = 5000
NB = N_MEMORY // BM
CHB = (0, 1280, 2560, 3840, 5000)
NCH = len(CHB) - 1
CPB = NCH * TOPK
LPB = 128
WPAD = 8
NEG = -1e30
BIG = 1 << 30
NWIN = WAY * TOPK


def _mean_support_dirs(inst_ref, rows):
    e25 = inst_ref[0:NSUP, :]
    ss = jnp.sum(e25 * e25, axis=1, keepdims=True)
    en = e25 / jnp.maximum(jnp.sqrt(ss), EPS)
    wio = jax.lax.broadcasted_iota(jnp.int32, (rows, NSUP), 0)
    jio = jax.lax.broadcasted_iota(jnp.int32, (rows, NSUP), 1)
    msel = jnp.where(jio % WAY == wio, 1.0 / SHOT, 0.0)
    a = jax.lax.dot_general(msel, en, (((1,), (0,)), ((), ())),
                            preferred_element_type=jnp.float32)
    return e25, en, a, wio, jio


def _scan_kernel(inst_ref, mem_ref, vals_ref, idx_ref):
    i = pl.program_id(0)
    _, _, a, _, _ = _mean_support_dirs(inst_ref, WAY)
    ones = jnp.ones((1, EMB_DIM), jnp.float32)
    vlist, ilist = [], []
    for q in range(NCH):
        s0, e0 = CHB[q], CHB[q + 1]
        ln = e0 - s0
        xc = mem_ref[s0:e0, :]
        sc = jax.lax.dot_general(a, xc, (((1,), (1,)), ((), ())),
                                 preferred_element_type=jnp.float32)
        sqc = jax.lax.dot_general(ones, xc * xc, (((1,), (1,)), ((), ())),
                                  preferred_element_type=jnp.float32)
        rnc = 1.0 / jnp.maximum(jnp.sqrt(sqc), EPS)
        sim = sc * rnc
        lio = jax.lax.broadcasted_iota(jnp.int32, (WAY, ln), 1)
        cur = sim
        mxs = []
        for _ in range(TOPK):
            mx = jnp.max(cur, axis=1, keepdims=True)
            mxs.append(mx)
            cur = jnp.where(cur == mx, NEG, cur)
        for mx in mxs:
            p = jnp.min(jnp.where(sim == mx, lio, BIG), axis=1, keepdims=True)
            vlist.append(mx)
            ilist.append(p + (i * BM + s0))
    vlist.append(jnp.full((WAY, LPB - CPB), NEG, jnp.float32))
    ilist.append(jnp.zeros((WAY, LPB - CPB), jnp.int32))
    vals = jnp.concatenate(vlist, axis=1)
    idx = jnp.concatenate(ilist, axis=1)
    vals_ref[0:WAY, :] = vals
    vals_ref[WAY:WPAD, :] = jnp.full((WPAD - WAY, LPB), NEG, jnp.float32)
    idx_ref[0:WAY, :] = idx
    idx_ref[WAY:WPAD, :] = jnp.zeros((WPAD - WAY, LPB), jnp.int32)


def _tail_kernel(vals_ref, idx_ref, inst_ref, mem_ref, out_ref,
                 rows_ref, sems_ref):
    e25, en, a, wio, jio = _mean_support_dirs(inst_ref, WPAD)
    g = jax.lax.dot_general(a, en, (((1,), (1,)), ((), ())),
                            preferred_element_type=jnp.float32)
    gm = jnp.where(jio % WAY == wio, g, NEG)
    cand = jnp.concatenate([gm, vals_ref[...]], axis=1)
    nc = NSUP + NB * LPB
    midxe = jnp.concatenate(
        [jnp.zeros((WPAD, NSUP), jnp.float32),
         idx_ref[...].astype(jnp.float32)], axis=1)
    lio = jax.lax.broadcasted_iota(jnp.int32, (WPAD, nc), 1)

    cur = cand
    mxs = []
    for _ in range(TOPK):
        mx = jnp.max(cur, axis=1, keepdims=True)
        mxs.append(mx)
        cur = jnp.where(cur == mx, NEG, cur)

    denom = mxs[0]
    for mx in mxs[1:]:
        denom = denom + mx
    rden = 1.0 / denom

    lio25 = jax.lax.broadcasted_iota(jnp.int32, (WPAD, NSUP), 1)
    wsup = jnp.zeros((WPAD, NSUP), jnp.float32)
    glist, wlist = [], []
    for mx in mxs:
        p = jnp.min(jnp.where(cand == mx, lio, BIG), axis=1, keepdims=True)
        is_mem = (p >= NSUP).astype(jnp.float32)
        glist.append(jnp.sum(jnp.where(lio == p, midxe, 0.0),
                             axis=1, keepdims=True))
        wlist.append(mx * is_mem)
        wsup = wsup + jnp.where(lio25 == p, mx, 0.0)

    gi = jnp.concatenate(glist, axis=1).astype(jnp.int32)
    wmn = jnp.concatenate(wlist, axis=1) * rden
    psup = jax.lax.dot_general(
        wsup * rden, e25, (((1,), (0,)), ((), ())),
        preferred_element_type=jnp.float32)

    def _cp(k):
        return pltpu.make_async_copy(
            mem_ref.at[pl.ds(gi[k // TOPK, k % TOPK], 1), :],
            rows_ref.at[pl.ds(k, 1), :],
            sems_ref.at[k])

    for k in range(NWIN):
        _cp(k).start()
    for k in range(NWIN):
        _cp(k).wait()

    parts = []
    for w in range(WAY):
        rw = rows_ref[w * TOPK:(w + 1) * TOPK, :]
        ww = wmn[w:w + 1, :]
        parts.append(jax.lax.dot_general(
            ww, rw, (((1,), (0,)), ((), ())),
            preferred_element_type=jnp.float32))
    p5 = psup[0:WAY, :] + jnp.concatenate(parts, axis=0)
    pn = p5 / jnp.maximum(
        jnp.sqrt(jnp.sum(p5 * p5, axis=1, keepdims=True)), EPS)
    q = inst_ref[NSUP:NSUP + NQ, :]
    out_ref[...] = jax.lax.dot_general(
        q, pn, (((1,), (1,)), ((), ())),
        preferred_element_type=jnp.float32) * (1.0 / TEMPERATURE)


def kernel(instance_embs, memory_bank):
    vals, idx = pl.pallas_call(
        _scan_kernel,
        grid=(NB,),
        in_specs=[
            pl.BlockSpec((NSUP + NQ, EMB_DIM), lambda i: (0, 0)),
            pl.BlockSpec((BM, EMB_DIM), lambda i: (i, 0)),
        ],
        out_specs=[
            pl.BlockSpec((WPAD, LPB), lambda i: (0, i)),
            pl.BlockSpec((WPAD, LPB), lambda i: (0, i)),
        ],
        out_shape=[
            jax.ShapeDtypeStruct((WPAD, NB * LPB), jnp.float32),
            jax.ShapeDtypeStruct((WPAD, NB * LPB), jnp.int32),
        ],
        compiler_params=pltpu.CompilerParams(
            dimension_semantics=("arbitrary",),
            vmem_limit_bytes=50 * 1024 * 1024,
        ),
        name="proto_scan",
    )(instance_embs, memory_bank)

    logits = pl.pallas_call(
        _tail_kernel,
        in_specs=[
            pl.BlockSpec(memory_space=pltpu.VMEM),
            pl.BlockSpec(memory_space=pltpu.VMEM),
            pl.BlockSpec(memory_space=pltpu.VMEM),
            pl.BlockSpec(memory_space=pl.ANY),
        ],
        out_shape=jax.ShapeDtypeStruct((NQ, WAY), jnp.float32),
        scratch_shapes=[
            pltpu.VMEM((NWIN, EMB_DIM), jnp.float32),
            pltpu.SemaphoreType.DMA((NWIN,)),
        ],
        name="proto_tail",
    )(vals, idx, instance_embs, memory_bank)

    return logits


# speedup vs baseline: 2.7536x; 1.1082x over previous
import jax
import jax.numpy as jnp
from jax.experimental import pallas as pl
from jax.experimental.pallas import tpu as pltpu

WAY = 5
SHOT = 5
QUERY = 15
EMB_DIM = 640
N_MEMORY = 100000
TEMPERATURE = 64.0
TOPK = 8
EPS = 1e-12

NSUP = WAY * SHOT
NQ = WAY * QUERY
BM = 5000
NB = N_MEMORY // BM
CHB = (0, 1280, 2560, 3840, 5000)
NCH = len(CHB) - 1
CPB = NCH * TOPK
LPB = 128
WPAD = 8
NEG = -1e30
BIG = 1 << 30
NWIN = WAY * TOPK
NCAND = NSUP + NB * LPB


def _mean_support_dirs(inst_ref, rows):
    e25 = inst_ref[0:NSUP, :]
    ss = jnp.sum(e25 * e25, axis=1, keepdims=True)
    en = e25 / jnp.maximum(jnp.sqrt(ss), EPS)
    wio = jax.lax.broadcasted_iota(jnp.int32, (rows, NSUP), 0)
    jio = jax.lax.broadcasted_iota(jnp.int32, (rows, NSUP), 1)
    msel = jnp.where(jio % WAY == wio, 1.0 / SHOT, 0.0)
    a = jax.lax.dot_general(msel, en, (((1,), (0,)), ((), ())),
                            preferred_element_type=jnp.float32)
    return e25, en, a, wio, jio


def _fused_kernel(inst_ref, mem_ref, memany_ref, out_ref,
                  a_sc, vals_sc, idx_sc, rows_sc, sems_ref):
    i = pl.program_id(0)

    @pl.when(i == 0)
    def _init():
        _, _, a8, _, _ = _mean_support_dirs(inst_ref, WPAD)
        a_sc[...] = a8

    a = a_sc[0:WAY, :]
    ones = jnp.ones((1, EMB_DIM), jnp.float32)
    vlist, ilist = [], []
    for q in range(NCH):
        s0, e0 = CHB[q], CHB[q + 1]
        ln = e0 - s0
        xc = mem_ref[s0:e0, :]
        sc = jax.lax.dot_general(a, xc, (((1,), (1,)), ((), ())),
                                 preferred_element_type=jnp.float32)
        sqc = jax.lax.dot_general(ones, xc * xc, (((1,), (1,)), ((), ())),
                                  preferred_element_type=jnp.float32)
        rnc = 1.0 / jnp.maximum(jnp.sqrt(sqc), EPS)
        sim = sc * rnc
        lio = jax.lax.broadcasted_iota(jnp.int32, (WAY, ln), 1)
        cur = sim
        mxs = []
        for _ in range(TOPK):
            mx = jnp.max(cur, axis=1, keepdims=True)
            mxs.append(mx)
            cur = jnp.where(cur == mx, NEG, cur)
        for mx in mxs:
            p = jnp.min(jnp.where(sim == mx, lio, BIG), axis=1, keepdims=True)
            vlist.append(mx)
            ilist.append(p + (i * BM + s0))
    vlist.append(jnp.full((WAY, LPB - CPB), NEG, jnp.float32))
    ilist.append(jnp.zeros((WAY, LPB - CPB), jnp.int32))
    vals = jnp.concatenate(
        [jnp.concatenate(vlist, axis=1),
         jnp.full((WPAD - WAY, LPB), NEG, jnp.float32)], axis=0)
    idx = jnp.concatenate(
        [jnp.concatenate(ilist, axis=1),
         jnp.zeros((WPAD - WAY, LPB), jnp.int32)], axis=0)
    off = pl.multiple_of(i * LPB, LPB)
    vals_sc[:, pl.ds(off, LPB)] = vals
    idx_sc[:, pl.ds(off, LPB)] = idx

    @pl.when(i == NB - 1)
    def _tail():
        e25, en, a8, wio, jio = _mean_support_dirs(inst_ref, WPAD)
        g = jax.lax.dot_general(a8, en, (((1,), (1,)), ((), ())),
                                preferred_element_type=jnp.float32)
        gm = jnp.where(jio % WAY == wio, g, NEG)
        cand = jnp.concatenate([gm, vals_sc[...]], axis=1)
        midxe = jnp.concatenate(
            [jnp.zeros((WPAD, NSUP), jnp.float32),
             idx_sc[...].astype(jnp.float32)], axis=1)
        lio = jax.lax.broadcasted_iota(jnp.int32, (WPAD, NCAND), 1)

        cur = cand
        mxs = []
        for _ in range(TOPK):
            mx = jnp.max(cur, axis=1, keepdims=True)
            mxs.append(mx)
            cur = jnp.where(cur == mx, NEG, cur)

        denom = mxs[0]
        for mx in mxs[1:]:
            denom = denom + mx
        rden = 1.0 / denom

        lio25 = jax.lax.broadcasted_iota(jnp.int32, (WPAD, NSUP), 1)
        wsup = jnp.zeros((WPAD, NSUP), jnp.float32)
        glist, wlist = [], []
        for mx in mxs:
            p = jnp.min(jnp.where(cand == mx, lio, BIG), axis=1, keepdims=True)
            is_mem = (p >= NSUP).astype(jnp.float32)
            glist.append(jnp.sum(jnp.where(lio == p, midxe, 0.0),
                                 axis=1, keepdims=True))
            wlist.append(mx * is_mem)
            wsup = wsup + jnp.where(lio25 == p, mx, 0.0)

        gi = jnp.concatenate(glist, axis=1).astype(jnp.int32)
        wmn = jnp.concatenate(wlist, axis=1) * rden
        psup = jax.lax.dot_general(
            wsup * rden, e25, (((1,), (0,)), ((), ())),
            preferred_element_type=jnp.float32)

        def _cp(k):
            return pltpu.make_async_copy(
                memany_ref.at[pl.ds(gi[k // TOPK, k % TOPK], 1), :],
                rows_sc.at[pl.ds(k, 1), :],
                sems_ref.at[k])

        for k in range(NWIN):
            _cp(k).start()
        for k in range(NWIN):
            _cp(k).wait()

        parts = []
        for w in range(WAY):
            rw = rows_sc[w * TOPK:(w + 1) * TOPK, :]
            ww = wmn[w:w + 1, :]
            parts.append(jax.lax.dot_general(
                ww, rw, (((1,), (0,)), ((), ())),
                preferred_element_type=jnp.float32))
        p5 = psup[0:WAY, :] + jnp.concatenate(parts, axis=0)
        pn = p5 / jnp.maximum(
            jnp.sqrt(jnp.sum(p5 * p5, axis=1, keepdims=True)), EPS)
        qm = inst_ref[NSUP:NSUP + NQ, :]
        out_ref[...] = jax.lax.dot_general(
            qm, pn, (((1,), (1,)), ((), ())),
            preferred_element_type=jnp.float32) * (1.0 / TEMPERATURE)


def kernel(instance_embs, memory_bank):
    return pl.pallas_call(
        _fused_kernel,
        grid=(NB,),
        in_specs=[
            pl.BlockSpec((NSUP + NQ, EMB_DIM), lambda i: (0, 0)),
            pl.BlockSpec((BM, EMB_DIM), lambda i: (i, 0)),
            pl.BlockSpec(memory_space=pl.ANY),
        ],
        out_specs=pl.BlockSpec((NQ, WAY), lambda i: (0, 0)),
        out_shape=jax.ShapeDtypeStruct((NQ, WAY), jnp.float32),
        scratch_shapes=[
            pltpu.VMEM((WPAD, EMB_DIM), jnp.float32),
            pltpu.VMEM((WPAD, NB * LPB), jnp.float32),
            pltpu.VMEM((WPAD, NB * LPB), jnp.int32),
            pltpu.VMEM((NWIN, EMB_DIM), jnp.float32),
            pltpu.SemaphoreType.DMA((NWIN,)),
        ],
        compiler_params=pltpu.CompilerParams(
            dimension_semantics=("arbitrary",),
            vmem_limit_bytes=50 * 1024 * 1024,
        ),
        name="proto_fused",
    )(instance_embs, memory_bank, memory_bank)
